# Initial kernel scaffold; baseline (speedup 1.0000x reference)
import jax
import jax.numpy as jnp
from jax.experimental import pallas as pl


def kernel(agent_feat, vox_feat, edge_src, edge_dst, Wq, bq, Wk, bk, Wv, bv, Wih, bih, Whh, bhh, Wo, bo, g1, bn1, g3, bn3, W1, bf1, W2, bf2):
    raise NotImplementedError("write your pallas kernel here")


# speedup vs baseline: 4.2095x; 4.2095x over previous
import functools
import jax
import jax.numpy as jnp
import numpy as np
from jax.experimental import pallas as pl

C = 128
NH = 8
DH = C // NH
NQ = 900
NV = 100 * 100 * 8
E = 600000
SP = 1024
TV = 320
NT = NV // TV


def _agent_kernel(xs_ref, g1_ref, bn1_ref, wk_ref, bk_ref, wv_ref, bv_ref,
                  k_ref, vm_ref):
    xs = xs_ref[...]
    m = jnp.mean(xs, axis=-1, keepdims=True)
    v = jnp.mean((xs - m) * (xs - m), axis=-1, keepdims=True)
    xsn = (xs - m) / jnp.sqrt(v + 1e-5) * g1_ref[0, :] + bn1_ref[0, :]
    mask = (jnp.sum(xsn, axis=-1, keepdims=True) != 0).astype(jnp.float32)
    k_ref[...] = jnp.dot(xsn, wk_ref[...], preferred_element_type=jnp.float32) + bk_ref[0, :]
    vv = jnp.dot(xsn, wv_ref[...], preferred_element_type=jnp.float32) + bv_ref[0, :]
    vm_ref[...] = vv * mask


def _voxel_kernel(xt_ref, cnt_ref, k_ref, vm_ref,
                  wq_ref, bq_ref, wih_ref, bih_ref, whh_ref, bhh_ref,
                  wo_ref, bo_ref, g3_ref, bn3_ref, w1_ref, bf1_ref,
                  w2_ref, bf2_ref, out_ref):
    xt = xt_ref[...]
    cnt = cnt_ref[...]
    q = jnp.dot(xt, wq_ref[...], preferred_element_type=jnp.float32) + bq_ref[0, :]
    kk = k_ref[...]
    vm = vm_ref[...]
    present = cnt > 0.0
    scale = 1.0 / np.sqrt(float(DH))
    aggs = []
    for h in range(NH):
        qh = q[:, h * DH:(h + 1) * DH]
        kh = kk[:, h * DH:(h + 1) * DH]
        ah = jax.lax.dot_general(qh, kh, (((1,), (1,)), ((), ())),
                                 preferred_element_type=jnp.float32) * scale
        amx = jnp.max(jnp.where(present, ah, -1e30), axis=1, keepdims=True)
        amx = jnp.where(amx > -1e29, amx, 0.0)
        w = cnt * jnp.exp(ah - amx)
        asum = jnp.sum(w, axis=1, keepdims=True)
        num = jnp.dot(w, vm[:, h * DH:(h + 1) * DH],
                      preferred_element_type=jnp.float32)
        aggs.append(num / (asum + 1e-16))
    agg = jnp.concatenate(aggs, axis=1)
    gate = jax.nn.sigmoid(
        jnp.dot(agg, wih_ref[...], preferred_element_type=jnp.float32) + bih_ref[0, :]
        + jnp.dot(xt, whh_ref[...], preferred_element_type=jnp.float32) + bhh_ref[0, :])
    upd = agg * gate
    mha = jnp.dot(upd, wo_ref[...], preferred_element_type=jnp.float32) + bo_ref[0, :]
    xt2 = xt + mha
    m = jnp.mean(xt2, axis=-1, keepdims=True)
    v = jnp.mean((xt2 - m) * (xt2 - m), axis=-1, keepdims=True)
    xtn = (xt2 - m) / jnp.sqrt(v + 1e-5) * g3_ref[0, :] + bn3_ref[0, :]
    ff = jnp.maximum(
        jnp.dot(xtn, w1_ref[...], preferred_element_type=jnp.float32) + bf1_ref[0, :], 0.0)
    ff = jnp.dot(ff, w2_ref[...], preferred_element_type=jnp.float32) + bf2_ref[0, :]
    out_ref[...] = xt2 + ff


@jax.jit
def kernel(agent_feat, vox_feat, edge_src, edge_dst, Wq, bq, Wk, bk, Wv, bv,
           Wih, bih, Whh, bhh, Wo, bo, g1, bn1, g3, bn3, W1, bf1, W2, bf2):
    xs = agent_feat.reshape(NQ, C)
    xt = vox_feat.reshape(NV, C)
    xsp = jnp.pad(xs, ((0, SP - NQ), (0, 0)))

    r2 = lambda a: a.reshape(1, -1)

    k_mat, vm_mat = pl.pallas_call(
        _agent_kernel,
        out_shape=[jax.ShapeDtypeStruct((SP, C), jnp.float32),
                   jax.ShapeDtypeStruct((SP, C), jnp.float32)],
    )(xsp, r2(g1), r2(bn1), Wk, r2(bk), Wv, r2(bv))

    cnt = jnp.zeros((NV, SP), jnp.float32).at[edge_dst, edge_src].add(1.0)

    bcast = pl.BlockSpec((SP, C), lambda i: (0, 0))
    wspec = lambda s: pl.BlockSpec(s, lambda i: (0, 0))
    out = pl.pallas_call(
        _voxel_kernel,
        grid=(NT,),
        in_specs=[
            pl.BlockSpec((TV, C), lambda i: (i, 0)),
            pl.BlockSpec((TV, SP), lambda i: (i, 0)),
            bcast,
            bcast,
            wspec((C, C)), wspec((1, C)),
            wspec((C, C)), wspec((1, C)),
            wspec((C, C)), wspec((1, C)),
            wspec((C, C)), wspec((1, C)),
            wspec((1, C)), wspec((1, C)),
            wspec((C, 4 * C)), wspec((1, 4 * C)),
            wspec((4 * C, C)), wspec((1, C)),
        ],
        out_specs=pl.BlockSpec((TV, C), lambda i: (i, 0)),
        out_shape=jax.ShapeDtypeStruct((NV, C), jnp.float32),
    )(xt, cnt, k_mat, vm_mat,
      Wq, r2(bq), Wih, r2(bih), Whh, r2(bhh), Wo, r2(bo),
      r2(g3), r2(bn3), W1, r2(bf1), W2, r2(bf2))

    return out.reshape(1, 100, 100, 8, C)


# speedup vs baseline: 4.6171x; 1.0968x over previous
import functools
import jax
import jax.numpy as jnp
import numpy as np
from jax.experimental import pallas as pl

C = 128
NH = 8
DH = C // NH
NQ = 900
NV = 100 * 100 * 8
E = 600000
SP = 1024
TV = 320
NT = NV // TV


def _agent_kernel(xs_ref, g1_ref, bn1_ref, wk_ref, bk_ref, wv_ref, bv_ref,
                  k_ref, vm_ref):
    xs = xs_ref[...]
    m = jnp.mean(xs, axis=-1, keepdims=True)
    v = jnp.mean((xs - m) * (xs - m), axis=-1, keepdims=True)
    xsn = (xs - m) / jnp.sqrt(v + 1e-5) * g1_ref[0, :] + bn1_ref[0, :]
    mask = (jnp.sum(xsn, axis=-1, keepdims=True) != 0).astype(jnp.float32)
    k_ref[...] = jnp.dot(xsn, wk_ref[...], preferred_element_type=jnp.float32) + bk_ref[0, :]
    vv = jnp.dot(xsn, wv_ref[...], preferred_element_type=jnp.float32) + bv_ref[0, :]
    vm_ref[...] = vv * mask


def _voxel_kernel(xt_ref, cnt_ref, k_ref, vm_ref,
                  wq_ref, bq_ref, wih_ref, bih_ref, whh_ref, bhh_ref,
                  wo_ref, bo_ref, g3_ref, bn3_ref, w1_ref, bf1_ref,
                  w2_ref, bf2_ref, out_ref):
    xt = xt_ref[...]
    cnt = cnt_ref[...]
    q = jnp.dot(xt, wq_ref[...], preferred_element_type=jnp.float32) + bq_ref[0, :]
    kk = k_ref[...].astype(jnp.bfloat16)
    vm = vm_ref[...].astype(jnp.bfloat16)
    qb = q.astype(jnp.bfloat16)
    cntb = cnt.astype(jnp.bfloat16)
    scale = np.float32(1.0 / np.sqrt(float(DH)))
    aggs = []
    for h in range(NH):
        qh = qb[:, h * DH:(h + 1) * DH]
        kh = kk[:, h * DH:(h + 1) * DH]
        ah = jax.lax.dot_general(qh, kh, (((1,), (1,)), ((), ())),
                                 preferred_element_type=jnp.float32) * scale
        amx = jnp.max(ah, axis=1, keepdims=True)
        w = cntb * jnp.exp((ah - amx).astype(jnp.bfloat16))
        vme = jnp.concatenate(
            [vm[:, h * DH:(h + 1) * DH],
             jnp.ones((SP, 1), jnp.bfloat16)], axis=1)
        num = jnp.dot(w, vme, preferred_element_type=jnp.float32)
        aggs.append(num[:, :DH] / (num[:, DH:DH + 1] + 1e-16))
    agg = jnp.concatenate(aggs, axis=1)
    gate = jax.nn.sigmoid(
        jnp.dot(agg, wih_ref[...], preferred_element_type=jnp.float32) + bih_ref[0, :]
        + jnp.dot(xt, whh_ref[...], preferred_element_type=jnp.float32) + bhh_ref[0, :])
    upd = agg * gate
    mha = jnp.dot(upd, wo_ref[...], preferred_element_type=jnp.float32) + bo_ref[0, :]
    xt2 = xt + mha
    m = jnp.mean(xt2, axis=-1, keepdims=True)
    v = jnp.mean((xt2 - m) * (xt2 - m), axis=-1, keepdims=True)
    xtn = (xt2 - m) / jnp.sqrt(v + 1e-5) * g3_ref[0, :] + bn3_ref[0, :]
    ff = jnp.maximum(
        jnp.dot(xtn, w1_ref[...], preferred_element_type=jnp.float32) + bf1_ref[0, :], 0.0)
    ff = jnp.dot(ff, w2_ref[...], preferred_element_type=jnp.float32) + bf2_ref[0, :]
    out_ref[...] = xt2 + ff


@jax.jit
def kernel(agent_feat, vox_feat, edge_src, edge_dst, Wq, bq, Wk, bk, Wv, bv,
           Wih, bih, Whh, bhh, Wo, bo, g1, bn1, g3, bn3, W1, bf1, W2, bf2):
    xs = agent_feat.reshape(NQ, C)
    xt = vox_feat.reshape(NV, C)
    xsp = jnp.pad(xs, ((0, SP - NQ), (0, 0)))

    r2 = lambda a: a.reshape(1, -1)

    k_mat, vm_mat = pl.pallas_call(
        _agent_kernel,
        out_shape=[jax.ShapeDtypeStruct((SP, C), jnp.float32),
                   jax.ShapeDtypeStruct((SP, C), jnp.float32)],
    )(xsp, r2(g1), r2(bn1), Wk, r2(bk), Wv, r2(bv))

    cnt = jnp.zeros((NV, SP), jnp.float32).at[edge_dst, edge_src].add(1.0)

    bcast = pl.BlockSpec((SP, C), lambda i: (0, 0))
    wspec = lambda s: pl.BlockSpec(s, lambda i: (0, 0))
    out = pl.pallas_call(
        _voxel_kernel,
        grid=(NT,),
        in_specs=[
            pl.BlockSpec((TV, C), lambda i: (i, 0)),
            pl.BlockSpec((TV, SP), lambda i: (i, 0)),
            bcast,
            bcast,
            wspec((C, C)), wspec((1, C)),
            wspec((C, C)), wspec((1, C)),
            wspec((C, C)), wspec((1, C)),
            wspec((C, C)), wspec((1, C)),
            wspec((1, C)), wspec((1, C)),
            wspec((C, 4 * C)), wspec((1, 4 * C)),
            wspec((4 * C, C)), wspec((1, C)),
        ],
        out_specs=pl.BlockSpec((TV, C), lambda i: (i, 0)),
        out_shape=jax.ShapeDtypeStruct((NV, C), jnp.float32),
    )(xt, cnt, k_mat, vm_mat,
      Wq, r2(bq), Wih, r2(bih), Whh, r2(bhh), Wo, r2(bo),
      r2(g3), r2(bn3), W1, r2(bf1), W2, r2(bf2))

    return out.reshape(1, 100, 100, 8, C)
